# Initial kernel scaffold; baseline (speedup 1.0000x reference)
import jax
import jax.numpy as jnp
from jax.experimental import pallas as pl


def kernel(x, conv1_w, conv1_b, conv2_w, conv2_b, bn1_gamma, bn1_beta, bn2_gamma, bn2_beta):
    raise NotImplementedError("write your pallas kernel here")


# speedup vs baseline: 4.9480x; 4.9480x over previous
import jax
import jax.numpy as jnp
from jax import lax
from jax.experimental import pallas as pl
from jax.experimental.pallas import tpu as pltpu

K = 8
BN_EPS = 1e-5
TH = 64


def _box8_lanes(a):
    s = a[..., :-1] + a[..., 1:]
    s = s[..., :-2] + s[..., 2:]
    return s[..., :-4] + s[..., 4:]


def _box8_rows(a):
    s = a[..., :-1, :] + a[..., 1:, :]
    s = s[..., :-2, :] + s[..., 2:, :]
    return s[..., :-4, :] + s[..., 4:, :]


def _pool_chunk(xm_ref, xh_ref):
    xa = jnp.concatenate([xm_ref[0], xh_ref[0]], axis=1)
    xw = _box8_lanes(xa)
    p = _box8_rows(xw)[:, :TH, :]
    return p * (1.0 / (K * K))


def _row_mask(r_idx, rows_total, nrows):
    valid = rows_total - r_idx * TH
    rows = lax.broadcasted_iota(jnp.int32, (1, nrows, 1), 1)
    return jnp.where(rows < valid, 1.0, 0.0)


def _stats_kernel(xm_ref, xh_ref, m_ref, g_ref, s_ref, *, hc):
    r = pl.program_id(1)
    m_ref[0] = jnp.sum(xm_ref[0], axis=0) * (1.0 / xm_ref.shape[1])
    p = _pool_chunk(xm_ref, xh_ref)
    pm = p * _row_mask(r, hc, TH)
    gb = lax.dot_general(
        pm, pm, (((2,), (2,)), ((1,), (1,))),
        preferred_element_type=jnp.float32)
    g_ref[0, 0] = jnp.sum(gb, axis=0)
    s_ref[0, 0, 0] = jnp.sum(pm, axis=(1, 2))


def _count_kernel(xm_ref, xh_ref, w1_ref, b1_ref, w2_ref, z2_ref, st_ref, *, hc):
    r = pl.program_id(1)
    p = _pool_chunk(xm_ref, xh_ref)
    z1 = jnp.tensordot(w1_ref[...], p, axes=([1], [0]),
                       preferred_element_type=jnp.float32)
    y1 = jnp.maximum(z1 + b1_ref[...][:, :, None], 0.0)
    z2 = jnp.sum(y1 * w2_ref[...][:, :, None], axis=0)
    z2_ref[0] = z2
    zm = z2 * _row_mask(r, hc, TH)[0]
    ssum = jnp.sum(zm).reshape(1, 1)
    ssq = jnp.sum(zm * zm).reshape(1, 1)
    st_ref[0, 0] = jnp.concatenate([ssum, ssq], axis=1)


def _fold_kernel(par_ref, m_ref, z2_ref, c_ref, r_ref):
    m = m_ref[0]
    h, w = m.shape
    hc, wc = h - K + 1, w - K + 1
    c = jnp.max(m)
    e = jnp.exp(m - c)
    z = _box8_rows(_box8_lanes(e))
    cnt = jnp.maximum(par_ref[0] * z2_ref[0] + par_ref[1], 0.0)
    c_ref[0, 0] = cnt
    q = cnt / z
    zc = jnp.zeros((hc, K), jnp.float32)
    qp = jnp.concatenate([zc, q, zc], axis=1)
    zr = jnp.zeros((K, wc + 2 * K), jnp.float32)
    qp = jnp.concatenate([zr, qp, zr], axis=0)
    t = _box8_rows(_box8_lanes(qp))
    r_ref[0, 0] = e * t[1:h + 1, 1:w + 1]


def kernel(x, conv1_w, conv1_b, conv2_w, conv2_b,
           bn1_gamma, bn1_beta, bn2_gamma, bn2_beta):
    import functools
    b, ch, h, w = x.shape
    hc, wc = h - K + 1, w - K + 1
    nr = h // TH
    f32 = jnp.float32

    xm_spec = pl.BlockSpec((1, ch, TH, w), lambda i, r: (i, 0, r, 0))
    halo_cap = h // K - 1
    xh_spec = pl.BlockSpec(
        (1, ch, K, w),
        lambda i, r: (i, 0, jnp.minimum((r + 1) * (TH // K), halo_cap), 0))

    m, gp, sp = pl.pallas_call(
        functools.partial(_stats_kernel, hc=hc),
        out_shape=(
            jax.ShapeDtypeStruct((b, h, w), f32),
            jax.ShapeDtypeStruct((b, nr, ch, ch), f32),
            jax.ShapeDtypeStruct((b, nr, 1, ch), f32),
        ),
        grid=(b, nr),
        in_specs=[xm_spec, xh_spec],
        out_specs=(
            pl.BlockSpec((1, TH, w), lambda i, r: (i, r, 0)),
            pl.BlockSpec((1, 1, ch, ch), lambda i, r: (i, r, 0, 0)),
            pl.BlockSpec((1, 1, 1, ch), lambda i, r: (i, r, 0, 0)),
        ),
        compiler_params=pltpu.CompilerParams(
            dimension_semantics=("parallel", "parallel")),
        name="counting_stats",
    )(x, x)

    n = b * hc * wc
    g = jnp.sum(gp, axis=(0, 1)) / n
    mu_p = jnp.sum(sp, axis=(0, 1, 2)) / n
    mean1 = conv1_w @ mu_p
    e2 = jnp.sum((conv1_w @ g) * conv1_w, axis=1)
    var1 = e2 - mean1 * mean1
    a1 = bn1_gamma * lax.rsqrt(var1 + BN_EPS)
    w1f = a1[:, None] * conv1_w
    b1f = (bn1_beta - a1 * mean1).reshape(ch, 1)
    w2col = conv2_w.reshape(ch, 1)

    z2, st = pl.pallas_call(
        functools.partial(_count_kernel, hc=hc),
        out_shape=(
            jax.ShapeDtypeStruct((b, hc, wc), f32),
            jax.ShapeDtypeStruct((b, nr, 1, 2), f32),
        ),
        grid=(b, nr),
        in_specs=[
            xm_spec, xh_spec,
            pl.BlockSpec((ch, ch), lambda i, r: (0, 0)),
            pl.BlockSpec((ch, 1), lambda i, r: (0, 0)),
            pl.BlockSpec((ch, 1), lambda i, r: (0, 0)),
        ],
        out_specs=(
            pl.BlockSpec((1, TH, wc), lambda i, r: (i, r, 0)),
            pl.BlockSpec((1, 1, 1, 2), lambda i, r: (i, r, 0, 0)),
        ),
        compiler_params=pltpu.CompilerParams(
            dimension_semantics=("parallel", "parallel")),
        name="counting_head",
    )(x, x, w1f, b1f, w2col)

    mean2 = jnp.sum(st[..., 0]) / n
    var2 = jnp.sum(st[..., 1]) / n - mean2 * mean2
    a2 = bn2_gamma[0] * lax.rsqrt(var2 + BN_EPS)
    c2 = bn2_beta[0] - a2 * mean2
    par = jnp.stack([a2, c2])

    c_out, r_out = pl.pallas_call(
        _fold_kernel,
        out_shape=(
            jax.ShapeDtypeStruct((b, 1, hc, wc), f32),
            jax.ShapeDtypeStruct((b, 1, h, w), f32),
        ),
        grid=(b,),
        in_specs=[
            pl.BlockSpec(memory_space=pltpu.SMEM),
            pl.BlockSpec((1, h, w), lambda i: (i, 0, 0)),
            pl.BlockSpec((1, hc, wc), lambda i: (i, 0, 0)),
        ],
        out_specs=(
            pl.BlockSpec((1, 1, hc, wc), lambda i: (i, 0, 0, 0)),
            pl.BlockSpec((1, 1, h, w), lambda i: (i, 0, 0, 0)),
        ),
        compiler_params=pltpu.CompilerParams(
            dimension_semantics=("parallel",)),
        name="counting_fold",
    )(par, m, z2)

    return c_out, r_out


# speedup vs baseline: 8.4520x; 1.7082x over previous
import functools

import jax
import jax.numpy as jnp
from jax import lax
from jax.experimental import pallas as pl
from jax.experimental.pallas import tpu as pltpu

K = 8
BN_EPS = 1e-5
TH = 64
CH = 24576


def _box8_lanes(a):
    s = a[..., :-1] + a[..., 1:]
    s = s[..., :-2] + s[..., 2:]
    return s[..., :-4] + s[..., 4:]


def _box8_rows(a):
    s = a[..., :-1, :] + a[..., 1:, :]
    s = s[..., :-2, :] + s[..., 2:, :]
    return s[..., :-4, :] + s[..., 4:, :]


def _pool_kernel(xm_ref, xh_ref, m_ref, p_ref):
    m_ref[0] = jnp.sum(xm_ref[0], axis=0) * (1.0 / xm_ref.shape[1])
    xa = jnp.concatenate([xm_ref[0], xh_ref[0]], axis=1)
    xw = _box8_lanes(xa)
    p_ref[0] = _box8_rows(xw)[:, :TH, :] * (1.0 / (K * K))


def _lane_select(c_idx, n_flat, v):
    lanes = lax.broadcasted_iota(jnp.int32, (1, CH), 1)
    return jnp.where(c_idx * CH + lanes < n_flat, v, 0.0)


def _gram_kernel(p_ref, g_ref, s_ref, *, n_flat):
    pm = _lane_select(pl.program_id(1), n_flat, p_ref[0])
    g_ref[0, 0] = lax.dot_general(
        pm, pm, (((1,), (1,)), ((), ())),
        preferred_element_type=jnp.float32)
    s_ref[0, 0, 0] = jnp.sum(pm, axis=1)


def _head_kernel(p_ref, w1_ref, b1_ref, w2_ref, z2_ref, st_ref, *, n_flat):
    z1 = jnp.dot(w1_ref[...], p_ref[0],
                 preferred_element_type=jnp.float32)
    y1 = jnp.maximum(z1 + b1_ref[...], 0.0)
    z2 = jnp.dot(w2_ref[...], y1,
                 preferred_element_type=jnp.float32)
    z2_ref[0] = z2
    zm = _lane_select(pl.program_id(1), n_flat, z2)
    ssum = jnp.sum(zm).reshape(1, 1)
    ssq = jnp.sum(zm * zm).reshape(1, 1)
    st_ref[0, 0] = jnp.concatenate([ssum, ssq], axis=1)


def _fold_kernel(par_ref, m_ref, z2_ref, c_ref, r_ref):
    m = m_ref[0]
    h, w = m.shape
    hc, wc = h - K + 1, w - K + 1
    c = jnp.max(m)
    e = jnp.exp(m - c)
    z = _box8_rows(_box8_lanes(e))
    cnt = jnp.maximum(par_ref[0] * z2_ref[0] + par_ref[1], 0.0)
    c_ref[0, 0] = cnt
    q = cnt / z
    zc = jnp.zeros((hc, K), jnp.float32)
    qp = jnp.concatenate([zc, q, zc], axis=1)
    zr = jnp.zeros((K, wc + 2 * K), jnp.float32)
    qp = jnp.concatenate([zr, qp, zr], axis=0)
    t = _box8_rows(_box8_lanes(qp))
    r_ref[0, 0] = e * t[1:h + 1, 1:w + 1]


def kernel(x, conv1_w, conv1_b, conv2_w, conv2_b,
           bn1_gamma, bn1_beta, bn2_gamma, bn2_beta):
    b, ch, h, w = x.shape
    hc, wc = h - K + 1, w - K + 1
    nr = h // TH
    n_flat = hc * wc
    nc = -(-n_flat // CH)
    f32 = jnp.float32

    xm_spec = pl.BlockSpec((1, ch, TH, w), lambda i, r: (i, 0, r, 0))
    halo_cap = h // K - 1
    xh_spec = pl.BlockSpec(
        (1, ch, K, w),
        lambda i, r: (i, 0, jnp.minimum((r + 1) * (TH // K), halo_cap), 0))

    m, p = pl.pallas_call(
        _pool_kernel,
        out_shape=(
            jax.ShapeDtypeStruct((b, h, w), f32),
            jax.ShapeDtypeStruct((b, ch, hc, wc), f32),
        ),
        grid=(b, nr),
        in_specs=[xm_spec, xh_spec],
        out_specs=(
            pl.BlockSpec((1, TH, w), lambda i, r: (i, r, 0)),
            pl.BlockSpec((1, ch, TH, wc), lambda i, r: (i, 0, r, 0)),
        ),
        compiler_params=pltpu.CompilerParams(
            dimension_semantics=("parallel", "parallel")),
        name="counting_pool",
    )(x, x)

    pf = p.reshape(b, ch, n_flat)
    pf_spec = pl.BlockSpec((1, ch, CH), lambda i, c: (i, 0, c))

    gp, sp = pl.pallas_call(
        functools.partial(_gram_kernel, n_flat=n_flat),
        out_shape=(
            jax.ShapeDtypeStruct((b, nc, ch, ch), f32),
            jax.ShapeDtypeStruct((b, nc, 1, ch), f32),
        ),
        grid=(b, nc),
        in_specs=[pf_spec],
        out_specs=(
            pl.BlockSpec((1, 1, ch, ch), lambda i, c: (i, c, 0, 0)),
            pl.BlockSpec((1, 1, 1, ch), lambda i, c: (i, c, 0, 0)),
        ),
        compiler_params=pltpu.CompilerParams(
            dimension_semantics=("parallel", "parallel")),
        name="counting_gram",
    )(pf)

    n = b * n_flat
    g = jnp.sum(gp, axis=(0, 1)) / n
    mu_p = jnp.sum(sp, axis=(0, 1, 2)) / n
    mean1 = conv1_w @ mu_p
    e2 = jnp.sum((conv1_w @ g) * conv1_w, axis=1)
    var1 = e2 - mean1 * mean1
    a1 = bn1_gamma * lax.rsqrt(var1 + BN_EPS)
    w1f = a1[:, None] * conv1_w
    b1f = (bn1_beta - a1 * mean1).reshape(ch, 1)

    z2f, st = pl.pallas_call(
        functools.partial(_head_kernel, n_flat=n_flat),
        out_shape=(
            jax.ShapeDtypeStruct((b, 1, n_flat), f32),
            jax.ShapeDtypeStruct((b, nc, 1, 2), f32),
        ),
        grid=(b, nc),
        in_specs=[
            pf_spec,
            pl.BlockSpec((ch, ch), lambda i, c: (0, 0)),
            pl.BlockSpec((ch, 1), lambda i, c: (0, 0)),
            pl.BlockSpec((1, ch), lambda i, c: (0, 0)),
        ],
        out_specs=(
            pl.BlockSpec((1, 1, CH), lambda i, c: (i, 0, c)),
            pl.BlockSpec((1, 1, 1, 2), lambda i, c: (i, c, 0, 0)),
        ),
        compiler_params=pltpu.CompilerParams(
            dimension_semantics=("parallel", "parallel")),
        name="counting_head",
    )(pf, w1f, b1f, conv2_w)

    mean2 = jnp.sum(st[..., 0]) / n
    var2 = jnp.sum(st[..., 1]) / n - mean2 * mean2
    a2 = bn2_gamma[0] * lax.rsqrt(var2 + BN_EPS)
    c2 = bn2_beta[0] - a2 * mean2
    par = jnp.stack([a2, c2])
    z2 = z2f.reshape(b, hc, wc)

    c_out, r_out = pl.pallas_call(
        _fold_kernel,
        out_shape=(
            jax.ShapeDtypeStruct((b, 1, hc, wc), f32),
            jax.ShapeDtypeStruct((b, 1, h, w), f32),
        ),
        grid=(b,),
        in_specs=[
            pl.BlockSpec(memory_space=pltpu.SMEM),
            pl.BlockSpec((1, h, w), lambda i: (i, 0, 0)),
            pl.BlockSpec((1, hc, wc), lambda i: (i, 0, 0)),
        ],
        out_specs=(
            pl.BlockSpec((1, 1, hc, wc), lambda i: (i, 0, 0, 0)),
            pl.BlockSpec((1, 1, h, w), lambda i: (i, 0, 0, 0)),
        ),
        compiler_params=pltpu.CompilerParams(
            dimension_semantics=("parallel",)),
        name="counting_fold",
    )(par, m, z2)

    return c_out, r_out


# speedup vs baseline: 8.5666x; 1.0136x over previous
import jax
import jax.numpy as jnp
from jax import lax
from jax.experimental import pallas as pl
from jax.experimental.pallas import tpu as pltpu

K = 8
BN_EPS = 1e-5
TH = 64
CH = 24576


def _box8_lanes(a):
    s = a[..., :-1] + a[..., 1:]
    s = s[..., :-2] + s[..., 2:]
    return s[..., :-4] + s[..., 4:]


def _box8_rows(a):
    s = a[..., :-1, :] + a[..., 1:, :]
    s = s[..., :-2, :] + s[..., 2:, :]
    return s[..., :-4, :] + s[..., 4:, :]


def _pool_kernel(xm_ref, xh_ref, m_ref, p_ref):
    m_ref[0] = jnp.sum(xm_ref[0], axis=0) * (1.0 / xm_ref.shape[1])
    xa = jnp.concatenate([xm_ref[0], xh_ref[0]], axis=1)
    xw = _box8_lanes(xa)
    p_ref[0] = _box8_rows(xw)[:, :TH, :] * (1.0 / (K * K))


def _lane_select(c_idx, n_flat, v):
    lanes = lax.broadcasted_iota(jnp.int32, (1, CH), 1)
    return jnp.where(c_idx * CH + lanes < n_flat, v, 0.0)


def _gram_kernel(p_ref, g_ref, s_ref, *, n_flat):
    pm = _lane_select(pl.program_id(1), n_flat, p_ref[0])
    g_ref[0, 0] = lax.dot_general(
        pm, pm, (((1,), (1,)), ((), ())),
        preferred_element_type=jnp.float32)
    s_ref[0, 0, 0] = jnp.sum(pm, axis=1)


def _head_kernel(p_ref, gp_ref, sp_ref, w1_ref, ga_ref, be_ref, w2_ref,
                 z2_ref, st_ref, *, n_total):
    inv_n = 1.0 / n_total
    g = jnp.sum(gp_ref[...], axis=(0, 1)) * inv_n
    mu = jnp.sum(sp_ref[...], axis=(0, 1, 2)) * inv_n
    w1 = w1_ref[...]
    mean1 = jnp.sum(w1 * mu[None, :], axis=1)[:, None]
    e2 = jnp.sum(jnp.dot(w1, g, preferred_element_type=jnp.float32) * w1,
                 axis=1)[:, None]
    var1 = e2 - mean1 * mean1
    a1 = ga_ref[...] * lax.rsqrt(var1 + BN_EPS)
    w1f = a1 * w1
    b1f = be_ref[...] - a1 * mean1
    z1 = jnp.dot(w1f, p_ref[0],
                 preferred_element_type=jnp.float32)
    y1 = jnp.maximum(z1 + b1f, 0.0)
    z2 = jnp.dot(w2_ref[...], y1,
                 preferred_element_type=jnp.float32)
    z2_ref[0] = z2
    zm = _lane_select(pl.program_id(1), n_total // gp_ref.shape[0], z2)
    ssum = jnp.sum(zm).reshape(1, 1)
    ssq = jnp.sum(zm * zm).reshape(1, 1)
    st_ref[0, 0] = jnp.concatenate([ssum, ssq], axis=1)


def _fold_kernel(st_ref, g2_ref, m_ref, z2_ref, c_ref, r_ref, *, n_total):
    mean2 = jnp.sum(st_ref[..., 0]) / n_total
    var2 = jnp.sum(st_ref[..., 1]) / n_total - mean2 * mean2
    a2 = g2_ref[0, 0] * lax.rsqrt(var2 + BN_EPS)
    c2 = g2_ref[1, 0] - a2 * mean2
    m = m_ref[0]
    h, w = m.shape
    hc, wc = h - K + 1, w - K + 1
    cmax = jnp.max(m)
    e = jnp.exp(m - cmax)
    z = _box8_rows(_box8_lanes(e))
    cnt = jnp.maximum(a2 * z2_ref[0] + c2, 0.0)
    c_ref[0, 0] = cnt
    q = cnt / z
    zc = jnp.zeros((hc, K), jnp.float32)
    qp = jnp.concatenate([zc, q, zc], axis=1)
    zr = jnp.zeros((K, wc + 2 * K), jnp.float32)
    qp = jnp.concatenate([zr, qp, zr], axis=0)
    t = _box8_rows(_box8_lanes(qp))
    r_ref[0, 0] = e * t[1:h + 1, 1:w + 1]


def kernel(x, conv1_w, conv1_b, conv2_w, conv2_b,
           bn1_gamma, bn1_beta, bn2_gamma, bn2_beta):
    import functools
    b, ch, h, w = x.shape
    hc, wc = h - K + 1, w - K + 1
    nr = h // TH
    n_flat = hc * wc
    nc = -(-n_flat // CH)
    n_total = b * n_flat
    f32 = jnp.float32

    xm_spec = pl.BlockSpec((1, ch, TH, w), lambda i, r: (i, 0, r, 0))
    halo_cap = h // K - 1
    xh_spec = pl.BlockSpec(
        (1, ch, K, w),
        lambda i, r: (i, 0, jnp.minimum((r + 1) * (TH // K), halo_cap), 0))

    m, p = pl.pallas_call(
        _pool_kernel,
        out_shape=(
            jax.ShapeDtypeStruct((b, h, w), f32),
            jax.ShapeDtypeStruct((b, ch, hc, wc), f32),
        ),
        grid=(b, nr),
        in_specs=[xm_spec, xh_spec],
        out_specs=(
            pl.BlockSpec((1, TH, w), lambda i, r: (i, r, 0)),
            pl.BlockSpec((1, ch, TH, wc), lambda i, r: (i, 0, r, 0)),
        ),
        compiler_params=pltpu.CompilerParams(
            dimension_semantics=("parallel", "parallel")),
        name="counting_pool",
    )(x, x)

    pf = p.reshape(b, ch, n_flat)
    pf_spec = pl.BlockSpec((1, ch, CH), lambda i, c: (i, 0, c))

    gp, sp = pl.pallas_call(
        functools.partial(_gram_kernel, n_flat=n_flat),
        out_shape=(
            jax.ShapeDtypeStruct((b, nc, ch, ch), f32),
            jax.ShapeDtypeStruct((b, nc, 1, ch), f32),
        ),
        grid=(b, nc),
        in_specs=[pf_spec],
        out_specs=(
            pl.BlockSpec((1, 1, ch, ch), lambda i, c: (i, c, 0, 0)),
            pl.BlockSpec((1, 1, 1, ch), lambda i, c: (i, c, 0, 0)),
        ),
        compiler_params=pltpu.CompilerParams(
            dimension_semantics=("parallel", "parallel")),
        name="counting_gram",
    )(pf)

    ga = bn1_gamma.reshape(ch, 1)
    be = bn1_beta.reshape(ch, 1)

    z2f, st = pl.pallas_call(
        functools.partial(_head_kernel, n_total=n_total),
        out_shape=(
            jax.ShapeDtypeStruct((b, 1, n_flat), f32),
            jax.ShapeDtypeStruct((b, nc, 1, 2), f32),
        ),
        grid=(b, nc),
        in_specs=[
            pf_spec,
            pl.BlockSpec((b, nc, ch, ch), lambda i, c: (0, 0, 0, 0)),
            pl.BlockSpec((b, nc, 1, ch), lambda i, c: (0, 0, 0, 0)),
            pl.BlockSpec((ch, ch), lambda i, c: (0, 0)),
            pl.BlockSpec((ch, 1), lambda i, c: (0, 0)),
            pl.BlockSpec((ch, 1), lambda i, c: (0, 0)),
            pl.BlockSpec((1, ch), lambda i, c: (0, 0)),
        ],
        out_specs=(
            pl.BlockSpec((1, 1, CH), lambda i, c: (i, 0, c)),
            pl.BlockSpec((1, 1, 1, 2), lambda i, c: (i, c, 0, 0)),
        ),
        compiler_params=pltpu.CompilerParams(
            dimension_semantics=("parallel", "parallel")),
        name="counting_head",
    )(pf, gp, sp, conv1_w, ga, be, conv2_w)

    g2 = jnp.stack([bn2_gamma[0], bn2_beta[0]]).reshape(2, 1)
    z2 = z2f.reshape(b, hc, wc)

    c_out, r_out = pl.pallas_call(
        functools.partial(_fold_kernel, n_total=n_total),
        out_shape=(
            jax.ShapeDtypeStruct((b, 1, hc, wc), f32),
            jax.ShapeDtypeStruct((b, 1, h, w), f32),
        ),
        grid=(b,),
        in_specs=[
            pl.BlockSpec((b, nc, 1, 2), lambda i: (0, 0, 0, 0)),
            pl.BlockSpec((2, 1), lambda i: (0, 0)),
            pl.BlockSpec((1, h, w), lambda i: (i, 0, 0)),
            pl.BlockSpec((1, hc, wc), lambda i: (i, 0, 0)),
        ],
        out_specs=(
            pl.BlockSpec((1, 1, hc, wc), lambda i: (i, 0, 0, 0)),
            pl.BlockSpec((1, 1, h, w), lambda i: (i, 0, 0, 0)),
        ),
        compiler_params=pltpu.CompilerParams(
            dimension_semantics=("parallel",)),
        name="counting_fold",
    )(st, g2, m, z2)

    return c_out, r_out


# speedup vs baseline: 8.8524x; 1.0334x over previous
import jax
import jax.numpy as jnp
from jax import lax
from jax.experimental import pallas as pl
from jax.experimental.pallas import tpu as pltpu

K = 8
BN_EPS = 1e-5
TH = 80
CH = 49152


def _box8_lanes(a):
    s = a[..., :-1] + a[..., 1:]
    s = s[..., :-2] + s[..., 2:]
    return s[..., :-4] + s[..., 4:]


def _box8_rows(a):
    s = a[..., :-1, :] + a[..., 1:, :]
    s = s[..., :-2, :] + s[..., 2:, :]
    return s[..., :-4, :] + s[..., 4:, :]


def _pool_kernel(xm_ref, xh_ref, m_ref, p_ref):
    m_ref[0] = jnp.sum(xm_ref[0], axis=0) * (1.0 / xm_ref.shape[1])
    xa = jnp.concatenate([xm_ref[0], xh_ref[0]], axis=1)
    xw = _box8_lanes(xa)
    p_ref[0] = _box8_rows(xw)[:, :TH, :] * (1.0 / (K * K))


def _lane_select(c_idx, n_flat, v):
    lanes = lax.broadcasted_iota(jnp.int32, (1, CH), 1)
    return jnp.where(c_idx * CH + lanes < n_flat, v, 0.0)


def _gram_kernel(p_ref, g_ref, s_ref, *, n_flat):
    pm = _lane_select(pl.program_id(1), n_flat, p_ref[0])
    g_ref[0, 0] = lax.dot_general(
        pm, pm, (((1,), (1,)), ((), ())),
        preferred_element_type=jnp.float32)
    s_ref[0, 0, 0] = jnp.sum(pm, axis=1)


def _head_kernel(p_ref, gp_ref, sp_ref, w1_ref, ga_ref, be_ref, w2_ref,
                 z2_ref, st_ref, *, n_total):
    inv_n = 1.0 / n_total
    g = jnp.sum(gp_ref[...], axis=(0, 1)) * inv_n
    mu = jnp.sum(sp_ref[...], axis=(0, 1, 2)) * inv_n
    w1 = w1_ref[...]
    mean1 = jnp.sum(w1 * mu[None, :], axis=1)[:, None]
    e2 = jnp.sum(jnp.dot(w1, g, preferred_element_type=jnp.float32) * w1,
                 axis=1)[:, None]
    var1 = e2 - mean1 * mean1
    a1 = ga_ref[...] * lax.rsqrt(var1 + BN_EPS)
    w1f = a1 * w1
    b1f = be_ref[...] - a1 * mean1
    z1 = jnp.dot(w1f, p_ref[0],
                 preferred_element_type=jnp.float32)
    y1 = jnp.maximum(z1 + b1f, 0.0)
    z2 = jnp.dot(w2_ref[...], y1,
                 preferred_element_type=jnp.float32)
    z2_ref[0] = z2
    zm = _lane_select(pl.program_id(1), n_total // gp_ref.shape[0], z2)
    ssum = jnp.sum(zm).reshape(1, 1)
    ssq = jnp.sum(zm * zm).reshape(1, 1)
    st_ref[0, 0] = jnp.concatenate([ssum, ssq], axis=1)


def _fold_kernel(st_ref, g2_ref, m_ref, z2_ref, c_ref, r_ref, *, n_total):
    mean2 = jnp.sum(st_ref[..., 0]) / n_total
    var2 = jnp.sum(st_ref[..., 1]) / n_total - mean2 * mean2
    a2 = g2_ref[0, 0] * lax.rsqrt(var2 + BN_EPS)
    c2 = g2_ref[1, 0] - a2 * mean2
    m = m_ref[0]
    h, w = m.shape
    hc, wc = h - K + 1, w - K + 1
    cmax = jnp.max(m)
    e = jnp.exp(m - cmax)
    z = _box8_rows(_box8_lanes(e))
    cnt = jnp.maximum(a2 * z2_ref[0] + c2, 0.0)
    c_ref[0, 0] = cnt
    q = cnt / z
    zc = jnp.zeros((hc, K), jnp.float32)
    qp = jnp.concatenate([zc, q, zc], axis=1)
    zr = jnp.zeros((K, wc + 2 * K), jnp.float32)
    qp = jnp.concatenate([zr, qp, zr], axis=0)
    t = _box8_rows(_box8_lanes(qp))
    r_ref[0, 0] = e * t[1:h + 1, 1:w + 1]


def kernel(x, conv1_w, conv1_b, conv2_w, conv2_b,
           bn1_gamma, bn1_beta, bn2_gamma, bn2_beta):
    import functools
    b, ch, h, w = x.shape
    hc, wc = h - K + 1, w - K + 1
    nr = h // TH
    n_flat = hc * wc
    nc = -(-n_flat // CH)
    n_total = b * n_flat
    f32 = jnp.float32

    xm_spec = pl.BlockSpec((1, ch, TH, w), lambda i, r: (i, 0, r, 0))
    halo_cap = h // K - 1
    xh_spec = pl.BlockSpec(
        (1, ch, K, w),
        lambda i, r: (i, 0, jnp.minimum((r + 1) * (TH // K), halo_cap), 0))

    m, p = pl.pallas_call(
        _pool_kernel,
        out_shape=(
            jax.ShapeDtypeStruct((b, h, w), f32),
            jax.ShapeDtypeStruct((b, ch, hc, wc), f32),
        ),
        grid=(b, nr),
        in_specs=[xm_spec, xh_spec],
        out_specs=(
            pl.BlockSpec((1, TH, w), lambda i, r: (i, r, 0)),
            pl.BlockSpec((1, ch, TH, wc), lambda i, r: (i, 0, r, 0)),
        ),
        compiler_params=pltpu.CompilerParams(
            dimension_semantics=("parallel", "parallel")),
        name="counting_pool",
    )(x, x)

    pf = p.reshape(b, ch, n_flat)
    pf_spec = pl.BlockSpec((1, ch, CH), lambda i, c: (i, 0, c))

    gp, sp = pl.pallas_call(
        functools.partial(_gram_kernel, n_flat=n_flat),
        out_shape=(
            jax.ShapeDtypeStruct((b, nc, ch, ch), f32),
            jax.ShapeDtypeStruct((b, nc, 1, ch), f32),
        ),
        grid=(b, nc),
        in_specs=[pf_spec],
        out_specs=(
            pl.BlockSpec((1, 1, ch, ch), lambda i, c: (i, c, 0, 0)),
            pl.BlockSpec((1, 1, 1, ch), lambda i, c: (i, c, 0, 0)),
        ),
        compiler_params=pltpu.CompilerParams(
            dimension_semantics=("parallel", "parallel")),
        name="counting_gram",
    )(pf)

    ga = bn1_gamma.reshape(ch, 1)
    be = bn1_beta.reshape(ch, 1)

    z2f, st = pl.pallas_call(
        functools.partial(_head_kernel, n_total=n_total),
        out_shape=(
            jax.ShapeDtypeStruct((b, 1, n_flat), f32),
            jax.ShapeDtypeStruct((b, nc, 1, 2), f32),
        ),
        grid=(b, nc),
        in_specs=[
            pf_spec,
            pl.BlockSpec((b, nc, ch, ch), lambda i, c: (0, 0, 0, 0)),
            pl.BlockSpec((b, nc, 1, ch), lambda i, c: (0, 0, 0, 0)),
            pl.BlockSpec((ch, ch), lambda i, c: (0, 0)),
            pl.BlockSpec((ch, 1), lambda i, c: (0, 0)),
            pl.BlockSpec((ch, 1), lambda i, c: (0, 0)),
            pl.BlockSpec((1, ch), lambda i, c: (0, 0)),
        ],
        out_specs=(
            pl.BlockSpec((1, 1, CH), lambda i, c: (i, 0, c)),
            pl.BlockSpec((1, 1, 1, 2), lambda i, c: (i, c, 0, 0)),
        ),
        compiler_params=pltpu.CompilerParams(
            dimension_semantics=("parallel", "parallel")),
        name="counting_head",
    )(pf, gp, sp, conv1_w, ga, be, conv2_w)

    g2 = jnp.stack([bn2_gamma[0], bn2_beta[0]]).reshape(2, 1)
    z2 = z2f.reshape(b, hc, wc)

    c_out, r_out = pl.pallas_call(
        functools.partial(_fold_kernel, n_total=n_total),
        out_shape=(
            jax.ShapeDtypeStruct((b, 1, hc, wc), f32),
            jax.ShapeDtypeStruct((b, 1, h, w), f32),
        ),
        grid=(b,),
        in_specs=[
            pl.BlockSpec((b, nc, 1, 2), lambda i: (0, 0, 0, 0)),
            pl.BlockSpec((2, 1), lambda i: (0, 0)),
            pl.BlockSpec((1, h, w), lambda i: (i, 0, 0)),
            pl.BlockSpec((1, hc, wc), lambda i: (i, 0, 0)),
        ],
        out_specs=(
            pl.BlockSpec((1, 1, hc, wc), lambda i: (i, 0, 0, 0)),
            pl.BlockSpec((1, 1, h, w), lambda i: (i, 0, 0, 0)),
        ),
        compiler_params=pltpu.CompilerParams(
            dimension_semantics=("parallel",)),
        name="counting_fold",
    )(st, g2, m, z2)

    return c_out, r_out


# speedup vs baseline: 9.5712x; 1.0812x over previous
import jax
import jax.numpy as jnp
from jax import lax
from jax.experimental import pallas as pl
from jax.experimental.pallas import tpu as pltpu

K = 8
BN_EPS = 1e-5
TH = 80
CH = 49152


def _box8_lanes(a):
    s = a[..., :-1] + a[..., 1:]
    s = s[..., :-2] + s[..., 2:]
    return s[..., :-4] + s[..., 4:]


def _box8_rows(a):
    s = a[..., :-1, :] + a[..., 1:, :]
    s = s[..., :-2, :] + s[..., 2:, :]
    return s[..., :-4, :] + s[..., 4:, :]


def _pool_kernel(xm_ref, xh_ref, m_ref, p_ref):
    m_ref[0] = jnp.sum(xm_ref[0], axis=0) * (1.0 / xm_ref.shape[1])
    xa = jnp.concatenate([xm_ref[0], xh_ref[0]], axis=1)
    xr = _box8_rows(xa)[:, :TH, :]
    p_ref[0] = _box8_lanes(xr) * (1.0 / (K * K))


def _lane_select(c_idx, n_flat, v):
    lanes = lax.broadcasted_iota(jnp.int32, (1, CH), 1)
    return jnp.where(c_idx * CH + lanes < n_flat, v, 0.0)


def _gram_kernel(p_ref, g_ref, s_ref, *, n_flat):
    pm = _lane_select(pl.program_id(1), n_flat, p_ref[0])
    g_ref[0, 0] = lax.dot_general(
        pm, pm, (((1,), (1,)), ((), ())),
        preferred_element_type=jnp.float32)
    s_ref[0, 0, 0] = jnp.sum(pm, axis=1)


def _head_kernel(p_ref, gp_ref, sp_ref, w1_ref, ga_ref, be_ref, w2_ref,
                 z2_ref, st_ref, *, n_total):
    inv_n = 1.0 / n_total
    g = jnp.sum(gp_ref[...], axis=(0, 1)) * inv_n
    mu = jnp.sum(sp_ref[...], axis=(0, 1, 2)) * inv_n
    w1 = w1_ref[...]
    mean1 = jnp.sum(w1 * mu[None, :], axis=1)[:, None]
    e2 = jnp.sum(jnp.dot(w1, g, preferred_element_type=jnp.float32) * w1,
                 axis=1)[:, None]
    var1 = e2 - mean1 * mean1
    a1 = ga_ref[...] * lax.rsqrt(var1 + BN_EPS)
    w1f = a1 * w1
    b1f = be_ref[...] - a1 * mean1
    z1 = jnp.dot(w1f, p_ref[0],
                 preferred_element_type=jnp.float32)
    y1 = jnp.maximum(z1 + b1f, 0.0)
    z2 = jnp.dot(w2_ref[...], y1,
                 preferred_element_type=jnp.float32)
    z2_ref[0] = z2
    zm = _lane_select(pl.program_id(1), n_total // gp_ref.shape[0], z2)
    ssum = jnp.sum(zm).reshape(1, 1)
    ssq = jnp.sum(zm * zm).reshape(1, 1)
    st_ref[0, 0] = jnp.concatenate([ssum, ssq], axis=1)


def _fold_kernel(st_ref, g2_ref, m_ref, z2_ref, c_ref, r_ref, *, n_total):
    mean2 = jnp.sum(st_ref[..., 0]) / n_total
    var2 = jnp.sum(st_ref[..., 1]) / n_total - mean2 * mean2
    a2 = g2_ref[0, 0] * lax.rsqrt(var2 + BN_EPS)
    c2 = g2_ref[1, 0] - a2 * mean2
    m = m_ref[0]
    h, w = m.shape
    hc, wc = h - K + 1, w - K + 1
    cmax = jnp.max(m)
    e = jnp.exp(m - cmax)
    z = _box8_rows(_box8_lanes(e))
    cnt = jnp.maximum(a2 * z2_ref[0] + c2, 0.0)
    c_ref[0, 0] = cnt
    q = cnt / z
    zc = jnp.zeros((hc, K), jnp.float32)
    qp = jnp.concatenate([zc, q, zc], axis=1)
    zr = jnp.zeros((K, wc + 2 * K), jnp.float32)
    qp = jnp.concatenate([zr, qp, zr], axis=0)
    t = _box8_rows(_box8_lanes(qp))
    r_ref[0, 0] = e * t[1:h + 1, 1:w + 1]


def kernel(x, conv1_w, conv1_b, conv2_w, conv2_b,
           bn1_gamma, bn1_beta, bn2_gamma, bn2_beta):
    import functools
    b, ch, h, w = x.shape
    hc, wc = h - K + 1, w - K + 1
    nr = h // TH
    n_flat = hc * wc
    nc = -(-n_flat // CH)
    n_total = b * n_flat
    f32 = jnp.float32

    xm_spec = pl.BlockSpec((1, ch, TH, w), lambda i, r: (i, 0, r, 0))
    halo_cap = h // K - 1
    xh_spec = pl.BlockSpec(
        (1, ch, K, w),
        lambda i, r: (i, 0, jnp.minimum((r + 1) * (TH // K), halo_cap), 0))

    m, p = pl.pallas_call(
        _pool_kernel,
        out_shape=(
            jax.ShapeDtypeStruct((b, h, w), f32),
            jax.ShapeDtypeStruct((b, ch, hc, wc), f32),
        ),
        grid=(b, nr),
        in_specs=[xm_spec, xh_spec],
        out_specs=(
            pl.BlockSpec((1, TH, w), lambda i, r: (i, r, 0)),
            pl.BlockSpec((1, ch, TH, wc), lambda i, r: (i, 0, r, 0)),
        ),
        compiler_params=pltpu.CompilerParams(
            dimension_semantics=("parallel", "parallel")),
        name="counting_pool",
    )(x, x)

    pf = p.reshape(b, ch, n_flat)
    pf_spec = pl.BlockSpec((1, ch, CH), lambda i, c: (i, 0, c))

    gp, sp = pl.pallas_call(
        functools.partial(_gram_kernel, n_flat=n_flat),
        out_shape=(
            jax.ShapeDtypeStruct((b, nc, ch, ch), f32),
            jax.ShapeDtypeStruct((b, nc, 1, ch), f32),
        ),
        grid=(b, nc),
        in_specs=[pf_spec],
        out_specs=(
            pl.BlockSpec((1, 1, ch, ch), lambda i, c: (i, c, 0, 0)),
            pl.BlockSpec((1, 1, 1, ch), lambda i, c: (i, c, 0, 0)),
        ),
        compiler_params=pltpu.CompilerParams(
            dimension_semantics=("parallel", "parallel")),
        name="counting_gram",
    )(pf)

    ga = bn1_gamma.reshape(ch, 1)
    be = bn1_beta.reshape(ch, 1)

    z2f, st = pl.pallas_call(
        functools.partial(_head_kernel, n_total=n_total),
        out_shape=(
            jax.ShapeDtypeStruct((b, 1, n_flat), f32),
            jax.ShapeDtypeStruct((b, nc, 1, 2), f32),
        ),
        grid=(b, nc),
        in_specs=[
            pf_spec,
            pl.BlockSpec((b, nc, ch, ch), lambda i, c: (0, 0, 0, 0)),
            pl.BlockSpec((b, nc, 1, ch), lambda i, c: (0, 0, 0, 0)),
            pl.BlockSpec((ch, ch), lambda i, c: (0, 0)),
            pl.BlockSpec((ch, 1), lambda i, c: (0, 0)),
            pl.BlockSpec((ch, 1), lambda i, c: (0, 0)),
            pl.BlockSpec((1, ch), lambda i, c: (0, 0)),
        ],
        out_specs=(
            pl.BlockSpec((1, 1, CH), lambda i, c: (i, 0, c)),
            pl.BlockSpec((1, 1, 1, 2), lambda i, c: (i, c, 0, 0)),
        ),
        compiler_params=pltpu.CompilerParams(
            dimension_semantics=("parallel", "parallel")),
        name="counting_head",
    )(pf, gp, sp, conv1_w, ga, be, conv2_w)

    g2 = jnp.stack([bn2_gamma[0], bn2_beta[0]]).reshape(2, 1)
    z2 = z2f.reshape(b, hc, wc)

    c_out, r_out = pl.pallas_call(
        functools.partial(_fold_kernel, n_total=n_total),
        out_shape=(
            jax.ShapeDtypeStruct((b, 1, hc, wc), f32),
            jax.ShapeDtypeStruct((b, 1, h, w), f32),
        ),
        grid=(b,),
        in_specs=[
            pl.BlockSpec((b, nc, 1, 2), lambda i: (0, 0, 0, 0)),
            pl.BlockSpec((2, 1), lambda i: (0, 0)),
            pl.BlockSpec((1, h, w), lambda i: (i, 0, 0)),
            pl.BlockSpec((1, hc, wc), lambda i: (i, 0, 0)),
        ],
        out_specs=(
            pl.BlockSpec((1, 1, hc, wc), lambda i: (i, 0, 0, 0)),
            pl.BlockSpec((1, 1, h, w), lambda i: (i, 0, 0, 0)),
        ),
        compiler_params=pltpu.CompilerParams(
            dimension_semantics=("parallel",)),
        name="counting_fold",
    )(st, g2, m, z2)

    return c_out, r_out
